# baseline (speedup 1.0000x reference)
import functools

import jax
import jax.numpy as jnp
from jax import lax
from jax.experimental import pallas as pl
from jax.experimental.pallas import tpu as pltpu
from jax.experimental.pallas import tpu_sc as plsc

NN = 10000
EE = 160000
DD = 256
HH = 128
LL = 3
NC = 2
NS = 16
RPW = 632
NPAD = NS * RPW
KE = 100
KR = 125
KP = 128
CPS = 8
SUP = 10
NCH_NAR = EE // (NC * NS) // KE
GROW = NPAD - 8
BR = 400


def _sc_mesh():
    return plsc.VectorSubcoreMesh(core_axis_name="c", subcore_axis_name="s")


@functools.partial(
    pl.kernel,
    out_type=jax.ShapeDtypeStruct((NC, NPAD, HH), jnp.float32),
    mesh=_sc_mesh(),
    scratch_types=[
        pltpu.VMEM((NCH_NAR, KE), jnp.int32),
        pltpu.VMEM((KE, HH), jnp.float32),
        pltpu.SemaphoreType.DMA,
        pltpu.SemaphoreType.DMA,
        pltpu.VMEM_SHARED((NPAD, HH), jnp.float32),
    ],
)
def _deg_pass(to_hbm, ones_hbm, zrow_hbm, deg_out,
              tslab, ones_v, ssem0, ssem1, acc_sh):
    c = lax.axis_index("c")
    s = lax.axis_index("s")
    pltpu.sync_copy(to_hbm.at[c, s], tslab)
    pltpu.sync_copy(ones_hbm, ones_v)
    pltpu.sync_copy(zrow_hbm, acc_sh.at[pl.ds(s * RPW, RPW)])
    plsc.subcore_barrier()

    def body(j, carry):
        i0 = j * 2
        i1 = i0 + 1

        @pl.when(j > 0)
        def _():
            pltpu.make_async_copy(ones_v, acc_sh.at[tslab.at[i0]], ssem0).wait()
            pltpu.make_async_copy(ones_v, acc_sh.at[tslab.at[i1]], ssem1).wait()

        pltpu.async_copy(ones_v, acc_sh.at[tslab.at[i0]], ssem0, add=True)
        pltpu.async_copy(ones_v, acc_sh.at[tslab.at[i1]], ssem1, add=True)
        return carry

    lax.fori_loop(0, NCH_NAR // 2, body, 0)
    pltpu.make_async_copy(ones_v, acc_sh.at[tslab.at[NCH_NAR - 2]], ssem0).wait()
    pltpu.make_async_copy(ones_v, acc_sh.at[tslab.at[NCH_NAR - 1]], ssem1).wait()
    plsc.subcore_barrier()
    pltpu.sync_copy(acc_sh.at[pl.ds(s * RPW, RPW)],
                    deg_out.at[c, pl.ds(s * RPW, RPW)])


@functools.partial(
    pl.kernel,
    out_type=jax.ShapeDtypeStruct((NC, NPAD, HH), jnp.float32),
    mesh=_sc_mesh(),
    scratch_types=[
        pltpu.VMEM((NCH_NAR, KE), jnp.int32),
        pltpu.VMEM((NCH_NAR, KE), jnp.int32),
        pltpu.VMEM((KE, HH), jnp.float32),
        pltpu.VMEM((KE, HH), jnp.float32),
        pltpu.SemaphoreType.DMA,
        pltpu.SemaphoreType.DMA,
        pltpu.SemaphoreType.DMA,
        pltpu.SemaphoreType.DMA,
        pltpu.VMEM_SHARED((NPAD, HH), jnp.float32),
    ],
)
def _t_pass(fr_hbm, to_hbm, dis_hbm, zrow_hbm, t_out,
            fslab, tslab, rows0, rows1, gsem0, gsem1, ssem0, ssem1, acc_sh):
    c = lax.axis_index("c")
    s = lax.axis_index("s")
    pltpu.sync_copy(fr_hbm.at[c, s], fslab)
    pltpu.sync_copy(to_hbm.at[c, s], tslab)
    pltpu.sync_copy(zrow_hbm, acc_sh.at[pl.ds(s * RPW, RPW)])
    plsc.subcore_barrier()
    pltpu.async_copy(dis_hbm.at[fslab.at[0]], rows0, gsem0)
    pltpu.async_copy(dis_hbm.at[fslab.at[1]], rows1, gsem1)

    def body(j, carry):
        i0 = j * 2
        i1 = i0 + 1
        pltpu.make_async_copy(dis_hbm.at[fslab.at[i0]], rows0, gsem0).wait()
        pltpu.async_copy(rows0, acc_sh.at[tslab.at[i0]], ssem0, add=True)
        pltpu.make_async_copy(dis_hbm.at[fslab.at[i1]], rows1, gsem1).wait()
        pltpu.async_copy(rows1, acc_sh.at[tslab.at[i1]], ssem1, add=True)

        @pl.when(j < NCH_NAR // 2 - 1)
        def _():
            pltpu.make_async_copy(rows0, acc_sh.at[tslab.at[i0]], ssem0).wait()
            pltpu.async_copy(dis_hbm.at[fslab.at[i0 + 2]], rows0, gsem0)
            pltpu.make_async_copy(rows1, acc_sh.at[tslab.at[i1]], ssem1).wait()
            pltpu.async_copy(dis_hbm.at[fslab.at[i1 + 2]], rows1, gsem1)

        return carry

    lax.fori_loop(0, NCH_NAR // 2, body, 0)
    pltpu.make_async_copy(rows0, acc_sh.at[tslab.at[NCH_NAR - 2]], ssem0).wait()
    pltpu.make_async_copy(rows1, acc_sh.at[tslab.at[NCH_NAR - 1]], ssem1).wait()
    plsc.subcore_barrier()
    pltpu.sync_copy(acc_sh.at[pl.ds(s * RPW, RPW)],
                    t_out.at[c, pl.ds(s * RPW, RPW)])


@functools.partial(
    pl.kernel,
    out_type=jax.ShapeDtypeStruct((NC, NPAD, HH), jnp.float32),
    mesh=_sc_mesh(),
    scratch_types=[
        pltpu.VMEM((2, CPS, KP), jnp.int32),
        pltpu.VMEM((2, CPS, KP), jnp.int32),
        pltpu.VMEM((KP, HH), jnp.float32),
        pltpu.VMEM((KP, HH), jnp.float32),
        pltpu.SemaphoreType.DMA((2,)),
        pltpu.SemaphoreType.DMA((2,)),
        pltpu.SemaphoreType.DMA,
        pltpu.SemaphoreType.DMA,
        pltpu.SemaphoreType.DMA,
        pltpu.SemaphoreType.DMA,
        pltpu.VMEM_SHARED((NPAD, HH), jnp.float32),
    ],
)
def _row_pass(f2_hbm, to_hbm, ytab_hbm, zrow_hbm, agg_out,
              fs3, ts3, rows0, rows1, ifsem, itsem,
              gsem0, gsem1, ssem0, ssem1, acc_sh):
    c = lax.axis_index("c")
    s = lax.axis_index("s")
    rows = (rows0, rows1)
    gsem = (gsem0, gsem1)
    ssem = (ssem0, ssem1)

    pltpu.async_copy(f2_hbm.at[c, s, 0], fs3.at[0], ifsem.at[0])
    pltpu.async_copy(to_hbm.at[s, 0], ts3.at[0], itsem.at[0])
    pltpu.sync_copy(zrow_hbm, acc_sh.at[pl.ds(s * RPW, RPW)])
    plsc.subcore_barrier()
    pltpu.make_async_copy(f2_hbm.at[c, s, 0], fs3.at[0], ifsem.at[0]).wait()
    pltpu.make_async_copy(to_hbm.at[s, 0], ts3.at[0], itsem.at[0]).wait()
    for b in range(2):
        pltpu.async_copy(ytab_hbm.at[fs3.at[0, b]], rows[b], gsem[b])

    def super_body(u, carry):
        p = lax.rem(u, 2)
        q = 1 - p

        @pl.when(u + 1 < SUP)
        def _():
            pltpu.async_copy(f2_hbm.at[c, s, u + 1], fs3.at[q], ifsem.at[q])
            pltpu.async_copy(to_hbm.at[s, u + 1], ts3.at[q], itsem.at[q])

        @pl.when(u > 0)
        def _():
            pltpu.make_async_copy(to_hbm.at[s, u], ts3.at[p], itsem.at[p]).wait()

        for j in range(CPS):
            b = j % 2
            pltpu.make_async_copy(ytab_hbm.at[fs3.at[p, j]], rows[b],
                                  gsem[b]).wait()
            pltpu.async_copy(rows[b], acc_sh.at[ts3.at[p, j]], ssem[b],
                             add=True)
            pltpu.make_async_copy(rows[b], acc_sh.at[ts3.at[p, j]],
                                  ssem[b]).wait()
            if j < CPS - 2:
                pltpu.async_copy(ytab_hbm.at[fs3.at[p, j + 2]], rows[b],
                                 gsem[b])
            elif j == CPS - 2:
                @pl.when(u + 1 < SUP)
                def _():
                    pltpu.make_async_copy(f2_hbm.at[c, s, u + 1], fs3.at[q],
                                          ifsem.at[q]).wait()
                    pltpu.async_copy(ytab_hbm.at[fs3.at[q, 0]], rows[b],
                                     gsem[b])
            else:
                @pl.when(u + 1 < SUP)
                def _():
                    pltpu.async_copy(ytab_hbm.at[fs3.at[q, 1]], rows[b],
                                     gsem[b])

        return carry

    lax.fori_loop(0, SUP, super_body, 0)
    plsc.subcore_barrier()
    pltpu.sync_copy(acc_sh.at[pl.ds(s * RPW, RPW)],
                    agg_out.at[c, pl.ds(s * RPW, RPW)])


def _prep_body(deg2_ref, emb_ref, dis_ref, y0_ref):
    deg = deg2_ref[0, :, 0:1] + deg2_ref[1, :, 0:1]
    dis = jnp.where(deg > 0.0, lax.rsqrt(deg), 0.0)
    dis_ref[...] = jnp.broadcast_to(dis, (BR, HH))
    y = dis * emb_ref[...]
    y0_ref[0] = y[:, :HH]
    y0_ref[1] = y[:, HH:]


def _prep_call(deg2, emb0):
    return pl.pallas_call(
        _prep_body,
        grid=(NN // BR,),
        in_specs=[
            pl.BlockSpec((NC, BR, HH), lambda i: (0, i, 0)),
            pl.BlockSpec((BR, DD), lambda i: (i, 0)),
        ],
        out_specs=[
            pl.BlockSpec((BR, HH), lambda i: (i, 0)),
            pl.BlockSpec((NC, BR, HH), lambda i: (0, i, 0)),
        ],
        out_shape=[
            jax.ShapeDtypeStruct((NN, HH), jnp.float32),
            jax.ShapeDtypeStruct((NC, NN, HH), jnp.float32),
        ],
    )(deg2, emb0)


def _layer_body(x_ref, agg_ref, dis_ref, t2_ref, w1_ref, w2_ref,
                b1_ref, b2_ref, xo_ref, y_ref):
    dis = dis_ref[:, 0:1]
    t = t2_ref[0, :, 0:1] + t2_ref[1, :, 0:1]
    sv = dis * t
    agg_raw = jnp.concatenate([agg_ref[0], agg_ref[1]], axis=-1)
    x = x_ref[...]
    agg1 = dis * agg_raw
    h1 = agg1 + x
    h2 = x * agg1
    out = (jnp.dot(h1, w1_ref[...], preferred_element_type=jnp.float32)
           + jnp.dot(h2, w2_ref[...], preferred_element_type=jnp.float32)
           + sv * (b1_ref[...] + b2_ref[...]) + b1_ref[...])
    xo = jnp.where(out >= 0.0, out, 0.01 * out)
    xo_ref[...] = xo
    y = dis * xo
    y_ref[0] = y[:, :HH]
    y_ref[1] = y[:, HH:]


def _layer_call(x, agg, dis_b, t2, w1t, w2t, b1l, b2l):
    return pl.pallas_call(
        _layer_body,
        grid=(NN // BR,),
        in_specs=[
            pl.BlockSpec((BR, DD), lambda i: (i, 0)),
            pl.BlockSpec((NC, BR, HH), lambda i: (0, i, 0)),
            pl.BlockSpec((BR, HH), lambda i: (i, 0)),
            pl.BlockSpec((NC, BR, HH), lambda i: (0, i, 0)),
            pl.BlockSpec((DD, DD), lambda i: (0, 0)),
            pl.BlockSpec((DD, DD), lambda i: (0, 0)),
            pl.BlockSpec((1, DD), lambda i: (0, 0)),
            pl.BlockSpec((1, DD), lambda i: (0, 0)),
        ],
        out_specs=[
            pl.BlockSpec((BR, DD), lambda i: (i, 0)),
            pl.BlockSpec((NC, BR, HH), lambda i: (0, i, 0)),
        ],
        out_shape=[
            jax.ShapeDtypeStruct((NN, DD), jnp.float32),
            jax.ShapeDtypeStruct((NC, NN, HH), jnp.float32),
        ],
    )(x, agg, dis_b, t2, w1t, w2t, b1l, b2l)


def kernel(edge_index, edge_attrs, emb_weight, W1, b1, W2, b2):
    fr = edge_index[0]
    to = edge_index[1]
    nrow = NC * NS * SUP * CPS
    f2p = jnp.concatenate(
        [jnp.concatenate([fr, fr + NN + 8]).reshape(nrow, KR),
         jnp.zeros((nrow, KP - KR), jnp.int32)], axis=1,
    ).reshape(NC, NS, SUP, CPS, KP)
    npadrow = nrow // NC
    spread = (NN + (jnp.arange(npadrow * (KP - KR), dtype=jnp.int32)
                    % (NPAD - NN - 8))).reshape(npadrow, KP - KR)
    top = jnp.concatenate(
        [to.reshape(npadrow, KR), spread], axis=1,
    ).reshape(NS, SUP, CPS, KP)
    fr_nar = fr.reshape(NC, NS, NCH_NAR, KE)
    to_nar = to.reshape(NC, NS, NCH_NAR, KE)
    ones_slab = jnp.ones((KE, HH), jnp.float32)
    zrow = jnp.zeros((RPW, HH), jnp.float32)

    deg2 = _deg_pass(to_nar, ones_slab, zrow)
    dis_b, y0 = _prep_call(deg2, emb_weight)
    t2 = _t_pass(fr_nar, to_nar, dis_b, zrow)

    x = emb_weight
    embs = [emb_weight]
    y = y0
    for l in range(LL):
        ytab = jnp.concatenate(
            [y[0], jnp.zeros((8, HH), jnp.float32), y[1]], axis=0)
        agg = _row_pass(f2p, top, ytab, zrow)
        x, y = _layer_call(x, agg, dis_b, t2, W1[l].T, W2[l].T,
                           b1[l][None, :], b2[l][None, :])
        embs.append(x)
    out = jnp.concatenate(embs, axis=-1)
    return emb_weight, out


# speedup vs baseline: 1.0638x; 1.0638x over previous
import functools

import jax
import jax.numpy as jnp
from jax import lax
from jax.experimental import pallas as pl
from jax.experimental.pallas import tpu as pltpu
from jax.experimental.pallas import tpu_sc as plsc

NN = 10000
EE = 160000
DD = 256
HH = 128
LL = 3
NC = 2
NS = 16
RPW = 632
NPAD = NS * RPW
KE = 100
KR = 125
KP = 128
CPS = 8
SUP = 10
NCH_NAR = EE // (NC * NS) // KE
GROW = NPAD - 8
BR = 400


def _sc_mesh():
    return plsc.VectorSubcoreMesh(core_axis_name="c", subcore_axis_name="s")


@functools.partial(
    pl.kernel,
    out_type=jax.ShapeDtypeStruct((NC, NPAD, HH), jnp.float32),
    mesh=_sc_mesh(),
    scratch_types=[
        pltpu.VMEM((NCH_NAR, KE), jnp.int32),
        pltpu.VMEM((KE, HH), jnp.float32),
        pltpu.SemaphoreType.DMA,
        pltpu.SemaphoreType.DMA,
        pltpu.VMEM_SHARED((NPAD, HH), jnp.float32),
    ],
)
def _deg_pass(to_hbm, ones_hbm, zrow_hbm, deg_out,
              tslab, ones_v, ssem0, ssem1, acc_sh):
    c = lax.axis_index("c")
    s = lax.axis_index("s")
    pltpu.sync_copy(to_hbm.at[c, s], tslab)
    pltpu.sync_copy(ones_hbm, ones_v)
    pltpu.sync_copy(zrow_hbm, acc_sh.at[pl.ds(s * RPW, RPW)])
    plsc.subcore_barrier()

    def body(j, carry):
        i0 = j * 2
        i1 = i0 + 1

        @pl.when(j > 0)
        def _():
            pltpu.make_async_copy(ones_v, acc_sh.at[tslab.at[i0]], ssem0).wait()
            pltpu.make_async_copy(ones_v, acc_sh.at[tslab.at[i1]], ssem1).wait()

        pltpu.async_copy(ones_v, acc_sh.at[tslab.at[i0]], ssem0, add=True)
        pltpu.async_copy(ones_v, acc_sh.at[tslab.at[i1]], ssem1, add=True)
        return carry

    lax.fori_loop(0, NCH_NAR // 2, body, 0)
    pltpu.make_async_copy(ones_v, acc_sh.at[tslab.at[NCH_NAR - 2]], ssem0).wait()
    pltpu.make_async_copy(ones_v, acc_sh.at[tslab.at[NCH_NAR - 1]], ssem1).wait()
    plsc.subcore_barrier()
    pltpu.sync_copy(acc_sh.at[pl.ds(s * RPW, RPW)],
                    deg_out.at[c, pl.ds(s * RPW, RPW)])


@functools.partial(
    pl.kernel,
    out_type=jax.ShapeDtypeStruct((NC, NPAD, HH), jnp.float32),
    mesh=_sc_mesh(),
    scratch_types=[
        pltpu.VMEM((NCH_NAR, KE), jnp.int32),
        pltpu.VMEM((NCH_NAR, KE), jnp.int32),
        pltpu.VMEM((KE, HH), jnp.float32),
        pltpu.VMEM((KE, HH), jnp.float32),
        pltpu.SemaphoreType.DMA,
        pltpu.SemaphoreType.DMA,
        pltpu.SemaphoreType.DMA,
        pltpu.SemaphoreType.DMA,
        pltpu.VMEM_SHARED((NPAD, HH), jnp.float32),
    ],
)
def _t_pass(fr_hbm, to_hbm, dis_hbm, zrow_hbm, t_out,
            fslab, tslab, rows0, rows1, gsem0, gsem1, ssem0, ssem1, acc_sh):
    c = lax.axis_index("c")
    s = lax.axis_index("s")
    pltpu.sync_copy(fr_hbm.at[c, s], fslab)
    pltpu.sync_copy(to_hbm.at[c, s], tslab)
    pltpu.sync_copy(zrow_hbm, acc_sh.at[pl.ds(s * RPW, RPW)])
    plsc.subcore_barrier()
    pltpu.async_copy(dis_hbm.at[fslab.at[0]], rows0, gsem0)
    pltpu.async_copy(dis_hbm.at[fslab.at[1]], rows1, gsem1)

    def body(j, carry):
        i0 = j * 2
        i1 = i0 + 1
        pltpu.make_async_copy(dis_hbm.at[fslab.at[i0]], rows0, gsem0).wait()
        pltpu.async_copy(rows0, acc_sh.at[tslab.at[i0]], ssem0, add=True)
        pltpu.make_async_copy(dis_hbm.at[fslab.at[i1]], rows1, gsem1).wait()
        pltpu.async_copy(rows1, acc_sh.at[tslab.at[i1]], ssem1, add=True)

        @pl.when(j < NCH_NAR // 2 - 1)
        def _():
            pltpu.make_async_copy(rows0, acc_sh.at[tslab.at[i0]], ssem0).wait()
            pltpu.async_copy(dis_hbm.at[fslab.at[i0 + 2]], rows0, gsem0)
            pltpu.make_async_copy(rows1, acc_sh.at[tslab.at[i1]], ssem1).wait()
            pltpu.async_copy(dis_hbm.at[fslab.at[i1 + 2]], rows1, gsem1)

        return carry

    lax.fori_loop(0, NCH_NAR // 2, body, 0)
    pltpu.make_async_copy(rows0, acc_sh.at[tslab.at[NCH_NAR - 2]], ssem0).wait()
    pltpu.make_async_copy(rows1, acc_sh.at[tslab.at[NCH_NAR - 1]], ssem1).wait()
    plsc.subcore_barrier()
    pltpu.sync_copy(acc_sh.at[pl.ds(s * RPW, RPW)],
                    t_out.at[c, pl.ds(s * RPW, RPW)])


@functools.partial(
    pl.kernel,
    out_type=jax.ShapeDtypeStruct((NC, NPAD, HH), jnp.float32),
    mesh=_sc_mesh(),
    scratch_types=[
        pltpu.VMEM((2, CPS, KP), jnp.int32),
        pltpu.VMEM((2, CPS, KP), jnp.int32),
        pltpu.VMEM((KP, HH), jnp.float32),
        pltpu.VMEM((KP, HH), jnp.float32),
        pltpu.SemaphoreType.DMA((2,)),
        pltpu.SemaphoreType.DMA((2,)),
        pltpu.SemaphoreType.DMA,
        pltpu.SemaphoreType.DMA,
        pltpu.SemaphoreType.DMA,
        pltpu.SemaphoreType.DMA,
        pltpu.VMEM_SHARED((NPAD, HH), jnp.float32),
    ],
)
def _row_pass(f2_hbm, to_hbm, ytab_hbm, zrow_hbm, agg_out,
              fs3, ts3, rows0, rows1, ifsem, itsem,
              gsem0, gsem1, ssem0, ssem1, acc_sh):
    c = lax.axis_index("c")
    s = lax.axis_index("s")
    rows = (rows0, rows1)
    gsem = (gsem0, gsem1)
    ssem = (ssem0, ssem1)

    pltpu.async_copy(f2_hbm.at[c, s, 0], fs3.at[0], ifsem.at[0])
    pltpu.async_copy(to_hbm.at[s, 0], ts3.at[0], itsem.at[0])
    pltpu.sync_copy(zrow_hbm, acc_sh.at[pl.ds(s * RPW, RPW)])
    plsc.subcore_barrier()
    pltpu.make_async_copy(f2_hbm.at[c, s, 0], fs3.at[0], ifsem.at[0]).wait()
    pltpu.make_async_copy(to_hbm.at[s, 0], ts3.at[0], itsem.at[0]).wait()
    for b in range(2):
        pltpu.async_copy(ytab_hbm.at[fs3.at[0, b]], rows[b], gsem[b])

    def super_body(u, carry):
        p = lax.rem(u, 2)
        q = 1 - p

        @pl.when(u + 1 < SUP)
        def _():
            pltpu.async_copy(f2_hbm.at[c, s, u + 1], fs3.at[q], ifsem.at[q])
            pltpu.async_copy(to_hbm.at[s, u + 1], ts3.at[q], itsem.at[q])

        @pl.when(u > 0)
        def _():
            pltpu.make_async_copy(to_hbm.at[s, u], ts3.at[p], itsem.at[p]).wait()

        for jj in range(CPS // 2):
            j0 = 2 * jj
            j1 = j0 + 1
            pltpu.make_async_copy(ytab_hbm.at[fs3.at[p, j0]], rows[0],
                                  gsem[0]).wait()
            pltpu.async_copy(rows[0], acc_sh.at[ts3.at[p, j0]], ssem[0],
                             add=True)
            pltpu.make_async_copy(ytab_hbm.at[fs3.at[p, j1]], rows[1],
                                  gsem[1]).wait()
            pltpu.async_copy(rows[1], acc_sh.at[ts3.at[p, j1]], ssem[1],
                             add=True)
            if jj < CPS // 2 - 1:
                pltpu.make_async_copy(rows[0], acc_sh.at[ts3.at[p, j0]],
                                      ssem[0]).wait()
                pltpu.async_copy(ytab_hbm.at[fs3.at[p, j0 + 2]], rows[0],
                                 gsem[0])
                pltpu.make_async_copy(rows[1], acc_sh.at[ts3.at[p, j1]],
                                      ssem[1]).wait()
                pltpu.async_copy(ytab_hbm.at[fs3.at[p, j1 + 2]], rows[1],
                                 gsem[1])
            else:
                @pl.when(u + 1 < SUP)
                def _():
                    pltpu.make_async_copy(f2_hbm.at[c, s, u + 1], fs3.at[q],
                                          ifsem.at[q]).wait()
                    pltpu.make_async_copy(rows[0], acc_sh.at[ts3.at[p, j0]],
                                          ssem[0]).wait()
                    pltpu.async_copy(ytab_hbm.at[fs3.at[q, 0]], rows[0],
                                     gsem[0])
                    pltpu.make_async_copy(rows[1], acc_sh.at[ts3.at[p, j1]],
                                          ssem[1]).wait()
                    pltpu.async_copy(ytab_hbm.at[fs3.at[q, 1]], rows[1],
                                     gsem[1])

        return carry

    lax.fori_loop(0, SUP, super_body, 0)
    for b in range(2):
        pltpu.make_async_copy(rows[b], acc_sh.at[ts3.at[0, b]], ssem[b]).wait()
    plsc.subcore_barrier()
    pltpu.sync_copy(acc_sh.at[pl.ds(s * RPW, RPW)],
                    agg_out.at[c, pl.ds(s * RPW, RPW)])


def _prep_body(deg2_ref, emb_ref, dis_ref, y0_ref):
    deg = deg2_ref[0, :, 0:1] + deg2_ref[1, :, 0:1]
    dis = jnp.where(deg > 0.0, lax.rsqrt(deg), 0.0)
    dis_ref[...] = jnp.broadcast_to(dis, (BR, HH))
    y = dis * emb_ref[...]
    y0_ref[0] = y[:, :HH]
    y0_ref[1] = y[:, HH:]


def _prep_call(deg2, emb0):
    return pl.pallas_call(
        _prep_body,
        grid=(NN // BR,),
        in_specs=[
            pl.BlockSpec((NC, BR, HH), lambda i: (0, i, 0)),
            pl.BlockSpec((BR, DD), lambda i: (i, 0)),
        ],
        out_specs=[
            pl.BlockSpec((BR, HH), lambda i: (i, 0)),
            pl.BlockSpec((NC, BR, HH), lambda i: (0, i, 0)),
        ],
        out_shape=[
            jax.ShapeDtypeStruct((NN, HH), jnp.float32),
            jax.ShapeDtypeStruct((NC, NN, HH), jnp.float32),
        ],
    )(deg2, emb0)


def _layer_body(x_ref, agg_ref, dis_ref, t2_ref, w1_ref, w2_ref,
                b1_ref, b2_ref, xo_ref, y_ref):
    dis = dis_ref[:, 0:1]
    t = t2_ref[0, :, 0:1] + t2_ref[1, :, 0:1]
    sv = dis * t
    agg_raw = jnp.concatenate([agg_ref[0], agg_ref[1]], axis=-1)
    x = x_ref[...]
    agg1 = dis * agg_raw
    h1 = agg1 + x
    h2 = x * agg1
    out = (jnp.dot(h1, w1_ref[...], preferred_element_type=jnp.float32)
           + jnp.dot(h2, w2_ref[...], preferred_element_type=jnp.float32)
           + sv * (b1_ref[...] + b2_ref[...]) + b1_ref[...])
    xo = jnp.where(out >= 0.0, out, 0.01 * out)
    xo_ref[...] = xo
    y = dis * xo
    y_ref[0] = y[:, :HH]
    y_ref[1] = y[:, HH:]


def _layer_call(x, agg, dis_b, t2, w1t, w2t, b1l, b2l):
    return pl.pallas_call(
        _layer_body,
        grid=(NN // BR,),
        in_specs=[
            pl.BlockSpec((BR, DD), lambda i: (i, 0)),
            pl.BlockSpec((NC, BR, HH), lambda i: (0, i, 0)),
            pl.BlockSpec((BR, HH), lambda i: (i, 0)),
            pl.BlockSpec((NC, BR, HH), lambda i: (0, i, 0)),
            pl.BlockSpec((DD, DD), lambda i: (0, 0)),
            pl.BlockSpec((DD, DD), lambda i: (0, 0)),
            pl.BlockSpec((1, DD), lambda i: (0, 0)),
            pl.BlockSpec((1, DD), lambda i: (0, 0)),
        ],
        out_specs=[
            pl.BlockSpec((BR, DD), lambda i: (i, 0)),
            pl.BlockSpec((NC, BR, HH), lambda i: (0, i, 0)),
        ],
        out_shape=[
            jax.ShapeDtypeStruct((NN, DD), jnp.float32),
            jax.ShapeDtypeStruct((NC, NN, HH), jnp.float32),
        ],
    )(x, agg, dis_b, t2, w1t, w2t, b1l, b2l)


def kernel(edge_index, edge_attrs, emb_weight, W1, b1, W2, b2):
    fr = edge_index[0]
    to = edge_index[1]
    nrow = NC * NS * SUP * CPS
    f2p = jnp.concatenate(
        [jnp.concatenate([fr, fr + NN]).reshape(nrow, KR),
         jnp.zeros((nrow, KP - KR), jnp.int32)], axis=1,
    ).reshape(NC, NS, SUP, CPS, KP)
    npadrow = nrow // NC
    spread = (NN + (jnp.arange(npadrow * (KP - KR), dtype=jnp.int32)
                    % (NPAD - NN - 8))).reshape(npadrow, KP - KR)
    top = jnp.concatenate(
        [to.reshape(npadrow, KR), spread], axis=1,
    ).reshape(NS, SUP, CPS, KP)
    fr_nar = fr.reshape(NC, NS, NCH_NAR, KE)
    to_nar = to.reshape(NC, NS, NCH_NAR, KE)
    ones_slab = jnp.ones((KE, HH), jnp.float32)
    zrow = jnp.zeros((RPW, HH), jnp.float32)

    deg2 = _deg_pass(to_nar, ones_slab, zrow)
    dis_b, y0 = _prep_call(deg2, emb_weight)
    t2 = _t_pass(fr_nar, to_nar, dis_b, zrow)

    x = emb_weight
    embs = [emb_weight]
    y = y0
    for l in range(LL):
        agg = _row_pass(f2p, top, y.reshape(NC * NN, HH), zrow)
        x, y = _layer_call(x, agg, dis_b, t2, W1[l].T, W2[l].T,
                           b1[l][None, :], b2[l][None, :])
        embs.append(x)
    out = jnp.concatenate(embs, axis=-1)
    return emb_weight, out


# speedup vs baseline: 2.0310x; 1.9092x over previous
import functools

import jax
import jax.numpy as jnp
from jax import lax
from jax.experimental import pallas as pl
from jax.experimental.pallas import tpu as pltpu
from jax.experimental.pallas import tpu_sc as plsc

NN = 10000
EE = 160000
DD = 256
HH = 128
LL = 3
NC = 2
NS = 16
RPW = 632
NPAD = NS * RPW
KE = 100
KR = 100
KP = 100
CPS = 10
SUP = 10
NCH_NAR = EE // (NC * NS) // KE
GROW = NPAD - 8
BR = 400


def _sc_mesh():
    return plsc.VectorSubcoreMesh(core_axis_name="c", subcore_axis_name="s")


@functools.partial(
    pl.kernel,
    out_type=jax.ShapeDtypeStruct((NC, NPAD, HH), jnp.float32),
    mesh=_sc_mesh(),
    scratch_types=[
        pltpu.VMEM((NCH_NAR, KE), jnp.int32),
        pltpu.VMEM((KE, HH), jnp.float32),
        pltpu.SemaphoreType.DMA,
        pltpu.SemaphoreType.DMA,
        pltpu.VMEM_SHARED((NPAD, HH), jnp.float32),
    ],
)
def _deg_pass(to_hbm, ones_hbm, zrow_hbm, deg_out,
              tslab, ones_v, ssem0, ssem1, acc_sh):
    c = lax.axis_index("c")
    s = lax.axis_index("s")
    pltpu.sync_copy(to_hbm.at[c, s], tslab)
    pltpu.sync_copy(ones_hbm, ones_v)
    pltpu.sync_copy(zrow_hbm, acc_sh.at[pl.ds(s * RPW, RPW)])
    plsc.subcore_barrier()

    def body(j, carry):
        i0 = j * 2
        i1 = i0 + 1

        @pl.when(j > 0)
        def _():
            pltpu.make_async_copy(ones_v, acc_sh.at[tslab.at[i0]], ssem0).wait()
            pltpu.make_async_copy(ones_v, acc_sh.at[tslab.at[i1]], ssem1).wait()

        pltpu.async_copy(ones_v, acc_sh.at[tslab.at[i0]], ssem0, add=True)
        pltpu.async_copy(ones_v, acc_sh.at[tslab.at[i1]], ssem1, add=True)
        return carry

    lax.fori_loop(0, NCH_NAR // 2, body, 0)
    pltpu.make_async_copy(ones_v, acc_sh.at[tslab.at[NCH_NAR - 2]], ssem0).wait()
    pltpu.make_async_copy(ones_v, acc_sh.at[tslab.at[NCH_NAR - 1]], ssem1).wait()
    plsc.subcore_barrier()
    pltpu.sync_copy(acc_sh.at[pl.ds(s * RPW, RPW)],
                    deg_out.at[c, pl.ds(s * RPW, RPW)])


@functools.partial(
    pl.kernel,
    out_type=jax.ShapeDtypeStruct((NC, NPAD, HH), jnp.float32),
    mesh=_sc_mesh(),
    scratch_types=[
        pltpu.VMEM((NCH_NAR, KE), jnp.int32),
        pltpu.VMEM((NCH_NAR, KE), jnp.int32),
        pltpu.VMEM((KE, HH), jnp.float32),
        pltpu.VMEM((KE, HH), jnp.float32),
        pltpu.SemaphoreType.DMA,
        pltpu.SemaphoreType.DMA,
        pltpu.SemaphoreType.DMA,
        pltpu.SemaphoreType.DMA,
        pltpu.VMEM_SHARED((NPAD, HH), jnp.float32),
    ],
)
def _t_pass(fr_hbm, to_hbm, dis_hbm, zrow_hbm, t_out,
            fslab, tslab, rows0, rows1, gsem0, gsem1, ssem0, ssem1, acc_sh):
    c = lax.axis_index("c")
    s = lax.axis_index("s")
    pltpu.sync_copy(fr_hbm.at[c, s], fslab)
    pltpu.sync_copy(to_hbm.at[c, s], tslab)
    pltpu.sync_copy(zrow_hbm, acc_sh.at[pl.ds(s * RPW, RPW)])
    plsc.subcore_barrier()
    pltpu.async_copy(dis_hbm.at[fslab.at[0]], rows0, gsem0)
    pltpu.async_copy(dis_hbm.at[fslab.at[1]], rows1, gsem1)

    def body(j, carry):
        i0 = j * 2
        i1 = i0 + 1
        pltpu.make_async_copy(dis_hbm.at[fslab.at[i0]], rows0, gsem0).wait()
        pltpu.async_copy(rows0, acc_sh.at[tslab.at[i0]], ssem0, add=True)
        pltpu.make_async_copy(dis_hbm.at[fslab.at[i1]], rows1, gsem1).wait()
        pltpu.async_copy(rows1, acc_sh.at[tslab.at[i1]], ssem1, add=True)

        @pl.when(j < NCH_NAR // 2 - 1)
        def _():
            pltpu.make_async_copy(rows0, acc_sh.at[tslab.at[i0]], ssem0).wait()
            pltpu.async_copy(dis_hbm.at[fslab.at[i0 + 2]], rows0, gsem0)
            pltpu.make_async_copy(rows1, acc_sh.at[tslab.at[i1]], ssem1).wait()
            pltpu.async_copy(dis_hbm.at[fslab.at[i1 + 2]], rows1, gsem1)

        return carry

    lax.fori_loop(0, NCH_NAR // 2, body, 0)
    pltpu.make_async_copy(rows0, acc_sh.at[tslab.at[NCH_NAR - 2]], ssem0).wait()
    pltpu.make_async_copy(rows1, acc_sh.at[tslab.at[NCH_NAR - 1]], ssem1).wait()
    plsc.subcore_barrier()
    pltpu.sync_copy(acc_sh.at[pl.ds(s * RPW, RPW)],
                    t_out.at[c, pl.ds(s * RPW, RPW)])


@functools.partial(
    pl.kernel,
    out_type=jax.ShapeDtypeStruct((NC, NPAD, HH), jnp.float32),
    mesh=_sc_mesh(),
    scratch_types=[
        pltpu.VMEM((2, CPS, KP), jnp.int32),
        pltpu.VMEM((2, CPS, KP), jnp.int32),
        pltpu.VMEM((KP, HH), jnp.float32),
        pltpu.VMEM((KP, HH), jnp.float32),
        pltpu.SemaphoreType.DMA((2,)),
        pltpu.SemaphoreType.DMA((2,)),
        pltpu.SemaphoreType.DMA,
        pltpu.SemaphoreType.DMA,
        pltpu.SemaphoreType.DMA,
        pltpu.SemaphoreType.DMA,
        pltpu.VMEM_SHARED((NPAD, HH), jnp.float32),
    ],
)
def _row_pass(f2_hbm, to_hbm, ytab_hbm, zrow_hbm, agg_out,
              fs3, ts3, rows0, rows1, ifsem, itsem,
              gsem0, gsem1, ssem0, ssem1, acc_sh):
    c = lax.axis_index("c")
    s = lax.axis_index("s")
    rows = (rows0, rows1)
    gsem = (gsem0, gsem1)
    ssem = (ssem0, ssem1)

    pltpu.async_copy(f2_hbm.at[c, s, 0], fs3.at[0], ifsem.at[0])
    pltpu.async_copy(to_hbm.at[s, 0], ts3.at[0], itsem.at[0])
    pltpu.sync_copy(zrow_hbm, acc_sh.at[pl.ds(s * RPW, RPW)])
    plsc.subcore_barrier()
    pltpu.make_async_copy(f2_hbm.at[c, s, 0], fs3.at[0], ifsem.at[0]).wait()
    pltpu.make_async_copy(to_hbm.at[s, 0], ts3.at[0], itsem.at[0]).wait()
    for b in range(2):
        pltpu.async_copy(ytab_hbm.at[fs3.at[0, b]], rows[b], gsem[b])

    def super_body(u, carry):
        p = lax.rem(u, 2)
        q = 1 - p

        @pl.when(u + 1 < SUP)
        def _():
            pltpu.async_copy(f2_hbm.at[c, s, u + 1], fs3.at[q], ifsem.at[q])
            pltpu.async_copy(to_hbm.at[s, u + 1], ts3.at[q], itsem.at[q])

        @pl.when(u > 0)
        def _():
            pltpu.make_async_copy(to_hbm.at[s, u], ts3.at[p], itsem.at[p]).wait()

        for jj in range(CPS // 2):
            j0 = 2 * jj
            j1 = j0 + 1
            pltpu.make_async_copy(ytab_hbm.at[fs3.at[p, j0]], rows[0],
                                  gsem[0]).wait()
            pltpu.async_copy(rows[0], acc_sh.at[ts3.at[p, j0]], ssem[0],
                             add=True)
            pltpu.make_async_copy(ytab_hbm.at[fs3.at[p, j1]], rows[1],
                                  gsem[1]).wait()
            pltpu.async_copy(rows[1], acc_sh.at[ts3.at[p, j1]], ssem[1],
                             add=True)
            if jj < CPS // 2 - 1:
                pltpu.make_async_copy(rows[0], acc_sh.at[ts3.at[p, j0]],
                                      ssem[0]).wait()
                pltpu.async_copy(ytab_hbm.at[fs3.at[p, j0 + 2]], rows[0],
                                 gsem[0])
                pltpu.make_async_copy(rows[1], acc_sh.at[ts3.at[p, j1]],
                                      ssem[1]).wait()
                pltpu.async_copy(ytab_hbm.at[fs3.at[p, j1 + 2]], rows[1],
                                 gsem[1])
            else:
                @pl.when(u + 1 < SUP)
                def _():
                    pltpu.make_async_copy(f2_hbm.at[c, s, u + 1], fs3.at[q],
                                          ifsem.at[q]).wait()
                    pltpu.make_async_copy(rows[0], acc_sh.at[ts3.at[p, j0]],
                                          ssem[0]).wait()
                    pltpu.async_copy(ytab_hbm.at[fs3.at[q, 0]], rows[0],
                                     gsem[0])
                    pltpu.make_async_copy(rows[1], acc_sh.at[ts3.at[p, j1]],
                                          ssem[1]).wait()
                    pltpu.async_copy(ytab_hbm.at[fs3.at[q, 1]], rows[1],
                                     gsem[1])

        return carry

    lax.fori_loop(0, SUP, super_body, 0)
    for b in range(2):
        pltpu.make_async_copy(rows[b], acc_sh.at[ts3.at[0, b]], ssem[b]).wait()
    plsc.subcore_barrier()
    pltpu.sync_copy(acc_sh.at[pl.ds(s * RPW, RPW)],
                    agg_out.at[c, pl.ds(s * RPW, RPW)])


def _prep_body(deg2_ref, emb_ref, dis_ref, y0_ref):
    deg = deg2_ref[0, :, 0:1] + deg2_ref[1, :, 0:1]
    dis = jnp.where(deg > 0.0, lax.rsqrt(deg), 0.0)
    dis_ref[...] = jnp.broadcast_to(dis, (BR, HH))
    y = dis * emb_ref[...]
    y0_ref[0] = y[:, :HH]
    y0_ref[1] = y[:, HH:]


def _prep_call(deg2, emb0):
    return pl.pallas_call(
        _prep_body,
        grid=(NN // BR,),
        in_specs=[
            pl.BlockSpec((NC, BR, HH), lambda i: (0, i, 0)),
            pl.BlockSpec((BR, DD), lambda i: (i, 0)),
        ],
        out_specs=[
            pl.BlockSpec((BR, HH), lambda i: (i, 0)),
            pl.BlockSpec((NC, BR, HH), lambda i: (0, i, 0)),
        ],
        out_shape=[
            jax.ShapeDtypeStruct((NN, HH), jnp.float32),
            jax.ShapeDtypeStruct((NC, NN, HH), jnp.float32),
        ],
    )(deg2, emb0)


def _layer_body(x_ref, agg_ref, dis_ref, t2_ref, w1_ref, w2_ref,
                b1_ref, b2_ref, xo_ref, y_ref):
    dis = dis_ref[:, 0:1]
    t = t2_ref[0, :, 0:1] + t2_ref[1, :, 0:1]
    sv = dis * t
    agg_raw = jnp.concatenate([agg_ref[0], agg_ref[1]], axis=-1)
    x = x_ref[...]
    agg1 = dis * agg_raw
    h1 = agg1 + x
    h2 = x * agg1
    out = (jnp.dot(h1, w1_ref[...], preferred_element_type=jnp.float32)
           + jnp.dot(h2, w2_ref[...], preferred_element_type=jnp.float32)
           + sv * (b1_ref[...] + b2_ref[...]) + b1_ref[...])
    xo = jnp.where(out >= 0.0, out, 0.01 * out)
    xo_ref[...] = xo
    y = dis * xo
    y_ref[0] = y[:, :HH]
    y_ref[1] = y[:, HH:]


def _layer_call(x, agg, dis_b, t2, w1t, w2t, b1l, b2l):
    return pl.pallas_call(
        _layer_body,
        grid=(NN // BR,),
        in_specs=[
            pl.BlockSpec((BR, DD), lambda i: (i, 0)),
            pl.BlockSpec((NC, BR, HH), lambda i: (0, i, 0)),
            pl.BlockSpec((BR, HH), lambda i: (i, 0)),
            pl.BlockSpec((NC, BR, HH), lambda i: (0, i, 0)),
            pl.BlockSpec((DD, DD), lambda i: (0, 0)),
            pl.BlockSpec((DD, DD), lambda i: (0, 0)),
            pl.BlockSpec((1, DD), lambda i: (0, 0)),
            pl.BlockSpec((1, DD), lambda i: (0, 0)),
        ],
        out_specs=[
            pl.BlockSpec((BR, DD), lambda i: (i, 0)),
            pl.BlockSpec((NC, BR, HH), lambda i: (0, i, 0)),
        ],
        out_shape=[
            jax.ShapeDtypeStruct((NN, DD), jnp.float32),
            jax.ShapeDtypeStruct((NC, NN, HH), jnp.float32),
        ],
    )(x, agg, dis_b, t2, w1t, w2t, b1l, b2l)


def kernel(edge_index, edge_attrs, emb_weight, W1, b1, W2, b2):
    fr = edge_index[0]
    to = edge_index[1]
    f2p = jnp.concatenate([fr, fr + NN]).reshape(NC, NS, SUP, CPS, KP)
    top = to.reshape(NS, SUP, CPS, KP)
    fr_nar = fr.reshape(NC, NS, NCH_NAR, KE)
    to_nar = to.reshape(NC, NS, NCH_NAR, KE)
    ones_slab = jnp.ones((KE, HH), jnp.float32)
    zrow = jnp.zeros((RPW, HH), jnp.float32)

    deg2 = _deg_pass(to_nar, ones_slab, zrow)
    dis_b, y0 = _prep_call(deg2, emb_weight)
    t2 = _t_pass(fr_nar, to_nar, dis_b, zrow)

    x = emb_weight
    embs = [emb_weight]
    y = y0
    for l in range(LL):
        agg = _row_pass(f2p, top, y.reshape(NC * NN, HH), zrow)
        x, y = _layer_call(x, agg, dis_b, t2, W1[l].T, W2[l].T,
                           b1[l][None, :], b2[l][None, :])
        embs.append(x)
    out = jnp.concatenate(embs, axis=-1)
    return emb_weight, out
